# baseline (speedup 1.0000x reference)
import functools

import jax
import jax.numpy as jnp
from jax import lax
from jax.experimental import pallas as pl
from jax.experimental.pallas import tpu as pltpu
from jax.experimental.pallas import tpu_sc as plsc

_NC = 2
_NS = 16
_NW = _NC * _NS

_NB = 4
_L = 16


def _emb(token_ids, Ec, batch, hist, d):
    ch = _NB * hist
    b_per_w = batch // _NW
    n_ch = b_per_w // _NB

    mesh = plsc.VectorSubcoreMesh(core_axis_name="c", subcore_axis_name="s")

    @functools.partial(
        pl.kernel,
        mesh=mesh,
        out_type=jax.ShapeDtypeStruct((batch, hist, d), jnp.float32),
        compiler_params=pltpu.CompilerParams(use_tc_tiling_on_sc=True),
        scratch_types=[
            pltpu.VMEM((_NB, hist), jnp.int32),
            pltpu.VMEM((ch,), jnp.int32),
            pltpu.VMEM((ch + _L,), jnp.int32),
            pltpu.VMEM((ch, 4 * d), jnp.float32),
            pltpu.VMEM((_NB, hist, d), jnp.float32),
            pltpu.SemaphoreType.DMA,
        ],
    )
    def emb_kernel(idx_hbm, table_hbm, out_hbm, idxbuf, q_ref, sel_v,
                   gbuf, obuf, gsem):
        wid = lax.axis_index("s") * _NC + lax.axis_index("c")
        w_b0 = wid * b_per_w

        offs = []
        o = 0
        while o + _L < hist:
            offs.append(o)
            o += _L
        offs.append(hist - _L)

        def chunk_body(c, _):
            b0 = w_b0 + c * _NB
            pltpu.sync_copy(idx_hbm.at[pl.ds(b0, _NB)], idxbuf)
            for r in range(_NB):
                for off in offs:
                    x = idxbuf[r, pl.ds(off, _L)]
                    q_ref[pl.ds(r * hist + off, _L)] = x >> 2
                    sel_v[pl.ds(r * hist + off, _L)] = (x & 3) << 5
            pltpu.async_copy(table_hbm.at[q_ref], gbuf, gsem).wait()

            def b_body(r, _):
                def h_body(h, _):
                    i = r * hist + h
                    cb = sel_v[pl.ds(i, _L)][0]
                    obuf[r, h, pl.ds(0, _L)] = gbuf[i, pl.ds(cb, _L)]
                    obuf[r, h, pl.ds(_L, _L)] = gbuf[i, pl.ds(cb + _L, _L)]
                    return ()
                lax.fori_loop(0, hist, h_body, ())
                return ()

            lax.fori_loop(0, _NB, b_body, ())
            pltpu.sync_copy(obuf, out_hbm.at[pl.ds(b0, _NB)])
            return ()

        lax.fori_loop(0, n_ch, chunk_body, ())

    return emb_kernel(token_ids, Ec)


def kernel(token_ids, E):
    batch, hist = token_ids.shape
    v, d = E.shape
    Ec = E.reshape(v // 4, 4 * d)
    return _emb(token_ids.astype(jnp.int32), Ec, batch, hist, d)


# speedup vs baseline: 1.3953x; 1.3953x over previous
import functools

import jax
import jax.numpy as jnp
from jax import lax
from jax.experimental import pallas as pl
from jax.experimental.pallas import tpu as pltpu
from jax.experimental.pallas import tpu_sc as plsc

_NC = 2
_NS = 16
_NW = _NC * _NS

_NB = 8
_NH = 4
_L = 16


def _emb(token_ids, Ec, batch, hist, d):
    ch = _NB * hist
    hh = _NH * hist
    b_per_w = batch // _NW
    n_ch = b_per_w // _NB

    mesh = plsc.VectorSubcoreMesh(core_axis_name="c", subcore_axis_name="s")

    offs = []
    o = 0
    while o + _L < hist:
        offs.append(o)
        o += _L
    offs.append(hist - _L)

    @functools.partial(
        pl.kernel,
        mesh=mesh,
        out_type=jax.ShapeDtypeStruct((batch, hist, d), jnp.float32),
        compiler_params=pltpu.CompilerParams(use_tc_tiling_on_sc=True),
        scratch_types=[
            pltpu.VMEM((_NB, hist), jnp.int32),
            pltpu.VMEM((hh,), jnp.int32),
            pltpu.VMEM((hh,), jnp.int32),
            pltpu.VMEM((ch + _L,), jnp.int32),
            pltpu.VMEM((hh, 4 * d), jnp.float32),
            pltpu.VMEM((hh, 4 * d), jnp.float32),
            pltpu.VMEM((_NB, hist, d), jnp.float32),
            pltpu.SemaphoreType.DMA,
            pltpu.SemaphoreType.DMA,
            pltpu.SemaphoreType.DMA,
        ],
    )
    def emb_kernel(idx_hbm, table_hbm, out_hbm, idxbuf, qA, qB, sel_v,
                   gbufA, gbufB, obuf, semA, semB, semO):
        wid = lax.axis_index("s") * _NC + lax.axis_index("c")
        w_b0 = wid * b_per_w

        def select_half(rlo, gval):
            def r_body(r, _):
                ibase = r * hist
                gbase = (r - rlo) * hist
                for h0 in offs:
                    sv = sel_v[pl.ds(ibase + h0, _L)]
                    for k in range(_L):
                        cb = sv[k]
                        gi = gbase + h0 + k
                        obuf[r, h0 + k, pl.ds(0, _L)] = \
                            gval[gi, pl.ds(cb, _L)]
                        obuf[r, h0 + k, pl.ds(_L, _L)] = \
                            gval[gi, pl.ds(cb + _L, _L)]
                return ()
            lax.fori_loop(rlo, rlo + _NH, r_body, ())

        def chunk_body(c, _):
            b0 = w_b0 + c * _NB
            pltpu.sync_copy(idx_hbm.at[pl.ds(b0, _NB)], idxbuf)
            for r in range(_NB):
                q_ref = qA if r < _NH else qB
                lbase = (r % _NH) * hist
                for off in offs:
                    x = idxbuf[r, pl.ds(off, _L)]
                    q_ref[pl.ds(lbase + off, _L)] = x >> 2
                    sel_v[pl.ds(r * hist + off, _L)] = (x & 3) << 5
            cpA = pltpu.async_copy(table_hbm.at[qA], gbufA, semA)
            cpB = pltpu.async_copy(table_hbm.at[qB], gbufB, semB)

            @pl.when(c > 0)
            def _():
                pltpu.make_async_copy(
                    obuf, out_hbm.at[pl.ds(b0, _NB)], semO).wait()

            cpA.wait()
            select_half(0, gbufA)
            cpB.wait()
            select_half(_NH, gbufB)
            pltpu.async_copy(obuf, out_hbm.at[pl.ds(b0, _NB)], semO)
            return ()

        lax.fori_loop(0, n_ch, chunk_body, ())
        pltpu.make_async_copy(
            obuf,
            out_hbm.at[pl.ds(w_b0 + (n_ch - 1) * _NB, _NB)],
            semO).wait()

    return emb_kernel(token_ids, Ec)


def kernel(token_ids, E):
    batch, hist = token_ids.shape
    v, d = E.shape
    Ec = E.reshape(v // 4, 4 * d)
    return _emb(token_ids.astype(jnp.int32), Ec, batch, hist, d)


# speedup vs baseline: 1.5074x; 1.0803x over previous
import functools

import jax
import jax.numpy as jnp
from jax import lax
from jax.experimental import pallas as pl
from jax.experimental.pallas import tpu as pltpu
from jax.experimental.pallas import tpu_sc as plsc

_NC = 2
_NS = 16
_NW = _NC * _NS

_NB = 8
_NH = 4
_L = 16


def _emb(token_ids, Ec, batch, hist, d):
    ch = _NB * hist
    hh = _NH * hist
    b_per_w = batch // _NW
    n_ch = b_per_w // _NB

    mesh = plsc.VectorSubcoreMesh(core_axis_name="c", subcore_axis_name="s")

    offs = []
    o = 0
    while o + _L < hist:
        offs.append(o)
        o += _L
    offs.append(hist - _L)

    @functools.partial(
        pl.kernel,
        mesh=mesh,
        out_type=jax.ShapeDtypeStruct((batch, hist, d), jnp.float32),
        compiler_params=pltpu.CompilerParams(use_tc_tiling_on_sc=True),
        scratch_types=[
            pltpu.VMEM((_NB, hist), jnp.int32),
            pltpu.VMEM((_NB, hist), jnp.int32),
            pltpu.VMEM((hh,), jnp.int32),
            pltpu.VMEM((hh,), jnp.int32),
            pltpu.VMEM((hh,), jnp.int32),
            pltpu.VMEM((hh,), jnp.int32),
            pltpu.VMEM((ch + _L,), jnp.int32),
            pltpu.VMEM((ch + _L,), jnp.int32),
            pltpu.VMEM((hh, 4 * d), jnp.float32),
            pltpu.VMEM((hh, 4 * d), jnp.float32),
            pltpu.VMEM((_NB, hist, d), jnp.float32),
            pltpu.SemaphoreType.DMA,
            pltpu.SemaphoreType.DMA,
            pltpu.SemaphoreType.DMA,
            pltpu.SemaphoreType.DMA,
        ],
    )
    def emb_kernel(idx_hbm, table_hbm, out_hbm,
                   idx0, idx1, qA0, qB0, qA1, qB1, sel0, sel1,
                   gbufA, gbufB, obuf, semI, semA, semB, semO):
        wid = lax.axis_index("s") * _NC + lax.axis_index("c")
        w_b0 = wid * b_per_w

        def build_q(idxbuf, qA, qB, sel_v):
            for r in range(_NB):
                q_ref = qA if r < _NH else qB
                lbase = (r % _NH) * hist
                for off in offs:
                    x = idxbuf[r, pl.ds(off, _L)]
                    q_ref[pl.ds(lbase + off, _L)] = x >> 2
                    sel_v[pl.ds(r * hist + off, _L)] = (x & 3) << 5

        def select_half(rlo, gval, sel_v):
            def r_body(r, _):
                ibase = r * hist
                gbase = (r - rlo) * hist
                for h0 in offs:
                    sv = sel_v[pl.ds(ibase + h0, _L)]
                    for k in range(_L):
                        cb = sv[k]
                        gi = gbase + h0 + k
                        obuf[r, h0 + k, pl.ds(0, _L)] = \
                            gval[gi, pl.ds(cb, _L)]
                        obuf[r, h0 + k, pl.ds(_L, _L)] = \
                            gval[gi, pl.ds(cb + _L, _L)]
                return ()
            lax.fori_loop(rlo, rlo + _NH, r_body, ())

        def phase(c, idx_cur, idx_nxt, q_cur, q_nxt, sel_cur, sel_nxt):
            b0 = w_b0 + c * _NB
            qAc, qBc = q_cur
            qAn, qBn = q_nxt

            @pl.when(c + 1 < n_ch)
            def _():
                pltpu.make_async_copy(
                    idx_hbm.at[pl.ds(b0 + _NB, _NB)], idx_nxt, semI).wait()
                build_q(idx_nxt, qAn, qBn, sel_nxt)

            @pl.when(c > 0)
            def _():
                pltpu.make_async_copy(
                    obuf, out_hbm.at[pl.ds(b0, _NB)], semO).wait()

            pltpu.make_async_copy(table_hbm.at[qAc], gbufA, semA).wait()
            select_half(0, gbufA, sel_cur)

            @pl.when(c + 1 < n_ch)
            def _():
                pltpu.async_copy(table_hbm.at[qAn], gbufA, semA)

            pltpu.make_async_copy(table_hbm.at[qBc], gbufB, semB).wait()
            select_half(_NH, gbufB, sel_cur)

            @pl.when(c + 1 < n_ch)
            def _():
                pltpu.async_copy(table_hbm.at[qBn], gbufB, semB)

            pltpu.async_copy(obuf, out_hbm.at[pl.ds(b0, _NB)], semO)

            @pl.when(c + 2 < n_ch)
            def _():
                pltpu.async_copy(
                    idx_hbm.at[pl.ds(b0 + 2 * _NB, _NB)], idx_cur, semI)

        pltpu.sync_copy(idx_hbm.at[pl.ds(w_b0, _NB)], idx0)
        build_q(idx0, qA0, qB0, sel0)
        pltpu.async_copy(table_hbm.at[qA0], gbufA, semA)
        pltpu.async_copy(table_hbm.at[qB0], gbufB, semB)
        pltpu.async_copy(idx_hbm.at[pl.ds(w_b0 + _NB, _NB)], idx1, semI)

        def pair_body(k, _):
            c = 2 * k
            phase(c, idx0, idx1, (qA0, qB0), (qA1, qB1), sel0, sel1)
            phase(c + 1, idx1, idx0, (qA1, qB1), (qA0, qB0), sel1, sel0)
            return ()

        lax.fori_loop(0, n_ch // 2, pair_body, ())
        pltpu.make_async_copy(
            obuf,
            out_hbm.at[pl.ds(w_b0 + (n_ch - 1) * _NB, _NB)],
            semO).wait()

    return emb_kernel(token_ids, Ec)


def kernel(token_ids, E):
    batch, hist = token_ids.shape
    v, d = E.shape
    Ec = E.reshape(v // 4, 4 * d)
    return _emb(token_ids.astype(jnp.int32), Ec, batch, hist, d)
